# baseline (speedup 1.0000x reference)
import functools

import jax
import jax.numpy as jnp
from jax import lax
from jax.experimental import pallas as pl
from jax.experimental.pallas import tpu as pltpu
from jax.experimental.pallas import tpu_sc as plsc

V = 100000
D = 16
TAU = 0.1
S = 1048576

NC = 2
NS = 16
NW = NC * NS
L = 16
PER_W = S // NW
CHUNK = 8192
NCHUNK = PER_W // CHUNK
UNROLL = 8
VPAD = 100096
VSLICE = VPAD // NS
GRID = 10
BLOCK_V = V // GRID

_CSCALE = 4095.0


def _prep_body(e0_ref, e1_ref, e2_ref, e3_ref, w2_ref, b_ref, lg_ref, o_ref):
    w2 = w2_ref[...]
    q = None
    for k, e_ref in enumerate((e0_ref, e1_ref, e2_ref, e3_ref)):
        part = lax.dot_general(
            w2[:, 4 * k:4 * k + 4], e_ref[...], (((1,), (0,)), ((), ())),
            preferred_element_type=jnp.float32,
        )
        q = part if q is None else q + part
    q = (q + b_ref[...]) * (1.0 / TAU)
    w = jnp.exp(q)
    wb = (lax.bitcast_convert_type(w, jnp.int32) + 0x800) & -4096
    c = 1.0 / (1.0 + jnp.exp(-lg_ref[...]))
    cu = (c * _CSCALE + 0.5).astype(jnp.int32)
    o_ref[pl.ds(0, V)] = wb.reshape(V) | cu


def _final_body(p_ref, cr_ref, o_ref):
    p = p_ref[...]
    den = jnp.sum(p[:, :L])
    num = jnp.sum(p[:, L:])
    explained = num / (jnp.maximum(den, 1e-30) * _CSCALE)
    explained = jnp.clip(explained, 0.0, 1.0)
    o_ref[...] = jnp.where(
        cr_ref[...] > 0.5,
        jax.nn.relu(1.0 - explained),
        jax.nn.relu(explained),
    )


def _sc_main(table_hbm, trace_hbm, out_hbm, table_v, buf0_v, buf1_v, res_v,
             table_sh, sem0, sem1):
    sid = lax.axis_index("s")
    wid = sid * NC + lax.axis_index("c")
    base = wid * PER_W
    bufs = [buf0_v, buf1_v]
    sems = [sem0, sem1]

    copies = [
        pltpu.async_copy(trace_hbm.at[pl.ds(base, CHUNK)], bufs[0], sems[0]),
        None,
    ]
    sl = pl.ds(sid * VSLICE, VSLICE)
    pltpu.sync_copy(table_hbm.at[sl], table_v.at[sl])
    pltpu.sync_copy(table_v.at[sl], table_sh.at[sl])
    plsc.subcore_barrier()
    pltpu.sync_copy(table_sh, table_v)

    mask12 = jnp.full((L,), 0xFFF, dtype=jnp.int32)
    maskw = jnp.full((L,), -4096, dtype=jnp.int32)
    carry = (jnp.zeros((L,), jnp.float32), jnp.zeros((L,), jnp.float32))

    for g in range(NCHUNK):
        nxt = g + 1
        if nxt < NCHUNK:
            copies[nxt % 2] = pltpu.async_copy(
                trace_hbm.at[pl.ds(base + nxt * CHUNK, CHUNK)],
                bufs[nxt % 2], sems[nxt % 2])
        copies[g % 2].wait()
        buf_v = bufs[g % 2]

        def body(j, car, buf_v=buf_v):
            den, num = car
            b = j * (L * UNROLL)
            for k in range(UNROLL):
                idx = buf_v[pl.ds(b + k * L, L)]
                packed = plsc.load_gather(table_v, [idx])
                w = plsc.bitcast(packed & maskw, jnp.float32)
                cf = (packed & mask12).astype(jnp.float32)
                den = den + w
                num = num + w * cf
            return (den, num)

        carry = lax.fori_loop(0, CHUNK // (L * UNROLL), body, carry)

    den, num = carry
    res_v[pl.ds(0, L)] = den
    res_v[pl.ds(L, L)] = num
    pltpu.sync_copy(res_v, out_hbm.at[wid])


def kernel(embed_table, attn_w, attn_b, causality_logits, event_trace, is_crash):
    et_t = embed_table.T
    b2 = attn_b.reshape(1, 1)

    packed = pl.pallas_call(
        _prep_body,
        out_shape=jax.ShapeDtypeStruct((VPAD,), jnp.int32),
    )(et_t[0:4], et_t[4:8], et_t[8:12], et_t[12:16],
      attn_w, b2, causality_logits)

    mesh = plsc.VectorSubcoreMesh(core_axis_name="c", subcore_axis_name="s")
    sc_call = functools.partial(
        pl.kernel,
        mesh=mesh,
        compiler_params=pltpu.CompilerParams(needs_layout_passes=False),
        out_type=jax.ShapeDtypeStruct((NW, 2 * L), jnp.float32),
        scratch_types=[
            pltpu.VMEM((VPAD,), jnp.int32),
            pltpu.VMEM((CHUNK,), jnp.int32),
            pltpu.VMEM((CHUNK,), jnp.int32),
            pltpu.VMEM((2 * L,), jnp.float32),
            pltpu.VMEM_SHARED((VPAD,), jnp.int32),
            pltpu.SemaphoreType.DMA,
            pltpu.SemaphoreType.DMA,
        ],
    )(_sc_main)
    partials = sc_call(packed, event_trace.astype(jnp.int32))

    crash2d = jnp.asarray(is_crash, jnp.float32).reshape(1, 1)
    res = pl.pallas_call(
        _final_body,
        out_shape=jax.ShapeDtypeStruct((1, 1), jnp.float32),
    )(partials, crash2d)
    return res[0, 0]


# speedup vs baseline: 1.2807x; 1.2807x over previous
import functools

import jax
import jax.numpy as jnp
from jax import lax
from jax.experimental import pallas as pl
from jax.experimental.pallas import tpu as pltpu
from jax.experimental.pallas import tpu_sc as plsc

V = 100000
D = 16
TAU = 0.1
S = 1048576

NC = 2
NS = 16
NW = NC * NS
L = 16
PER_W = S // NW
CHUNK = 8192
NCHUNK = PER_W // CHUNK
UNROLL = 8
VPAD = 100096
VSLICE = VPAD // NS
GRID = 10
BLOCK_V = V // GRID

_CSCALE = 4095.0


def _prep_body(e_ref, w2_ref, b_ref, lg_ref, o_ref):
    q = lax.dot_general(
        w2_ref[...], e_ref[...], (((1,), (0,)), ((), ())),
        preferred_element_type=jnp.float32,
    )
    q = (q + b_ref[...]) * (1.0 / TAU)
    w = jnp.exp(q)
    wb = (lax.bitcast_convert_type(w, jnp.int32) + 0x800) & -4096
    c = 1.0 / (1.0 + jnp.exp(-lg_ref[...]))
    cu = (c * _CSCALE + 0.5).astype(jnp.int32)
    o_ref[pl.ds(0, V)] = wb.reshape(V) | cu


def _final_body(p_ref, cr_ref, o_ref):
    p = p_ref[...]
    den = jnp.sum(p[:, :L])
    num = jnp.sum(p[:, L:])
    explained = num / (jnp.maximum(den, 1e-30) * _CSCALE)
    explained = jnp.clip(explained, 0.0, 1.0)
    o_ref[...] = jnp.where(
        cr_ref[...] > 0.5,
        jax.nn.relu(1.0 - explained),
        jax.nn.relu(explained),
    )


def _sc_main(table_hbm, trace_hbm, out_hbm, table_v, buf0_v, buf1_v, res_v,
             table_sh, sem0, sem1):
    sid = lax.axis_index("s")
    wid = sid * NC + lax.axis_index("c")
    base = wid * PER_W
    bufs = [buf0_v, buf1_v]
    sems = [sem0, sem1]

    copies = [
        pltpu.async_copy(trace_hbm.at[pl.ds(base, CHUNK)], bufs[0], sems[0]),
        None,
    ]
    sl = pl.ds(sid * VSLICE, VSLICE)
    pltpu.sync_copy(table_hbm.at[sl], table_v.at[sl])
    pltpu.sync_copy(table_v.at[sl], table_sh.at[sl])
    plsc.subcore_barrier()
    pltpu.sync_copy(table_sh, table_v)

    mask12 = jnp.full((L,), 0xFFF, dtype=jnp.int32)
    maskw = jnp.full((L,), -4096, dtype=jnp.int32)
    carry = (jnp.zeros((L,), jnp.float32), jnp.zeros((L,), jnp.float32))

    for g in range(NCHUNK):
        nxt = g + 1
        if nxt < NCHUNK:
            copies[nxt % 2] = pltpu.async_copy(
                trace_hbm.at[pl.ds(base + nxt * CHUNK, CHUNK)],
                bufs[nxt % 2], sems[nxt % 2])
        copies[g % 2].wait()
        buf_v = bufs[g % 2]

        def body(j, car, buf_v=buf_v):
            den, num = car
            b = j * (L * UNROLL)
            for k in range(UNROLL):
                idx = buf_v[pl.ds(b + k * L, L)]
                packed = plsc.load_gather(table_v, [idx])
                w = plsc.bitcast(packed & maskw, jnp.float32)
                cf = (packed & mask12).astype(jnp.float32)
                den = den + w
                num = num + w * cf
            return (den, num)

        carry = lax.fori_loop(0, CHUNK // (L * UNROLL), body, carry)

    den, num = carry
    res_v[pl.ds(0, L)] = den
    res_v[pl.ds(L, L)] = num
    pltpu.sync_copy(res_v, out_hbm.at[wid])


def kernel(embed_table, attn_w, attn_b, causality_logits, event_trace, is_crash):
    et_t = embed_table.T
    b2 = attn_b.reshape(1, 1)

    packed = pl.pallas_call(
        _prep_body,
        out_shape=jax.ShapeDtypeStruct((VPAD,), jnp.int32),
    )(et_t, attn_w, b2, causality_logits)

    mesh = plsc.VectorSubcoreMesh(core_axis_name="c", subcore_axis_name="s")
    sc_call = functools.partial(
        pl.kernel,
        mesh=mesh,
        compiler_params=pltpu.CompilerParams(needs_layout_passes=False),
        out_type=jax.ShapeDtypeStruct((NW, 2 * L), jnp.float32),
        scratch_types=[
            pltpu.VMEM((VPAD,), jnp.int32),
            pltpu.VMEM((CHUNK,), jnp.int32),
            pltpu.VMEM((CHUNK,), jnp.int32),
            pltpu.VMEM((2 * L,), jnp.float32),
            pltpu.VMEM_SHARED((VPAD,), jnp.int32),
            pltpu.SemaphoreType.DMA,
            pltpu.SemaphoreType.DMA,
        ],
    )(_sc_main)
    partials = sc_call(packed, event_trace.astype(jnp.int32))

    crash2d = jnp.asarray(is_crash, jnp.float32).reshape(1, 1)
    res = pl.pallas_call(
        _final_body,
        out_shape=jax.ShapeDtypeStruct((1, 1), jnp.float32),
    )(partials, crash2d)
    return res[0, 0]
